# Initial kernel scaffold; baseline (speedup 1.0000x reference)
import jax
import jax.numpy as jnp
from jax.experimental import pallas as pl


def kernel(x, sa1_w1, sa1_b1, sa1_w2, sa1_b2, sa2_w1, sa2_b1, sa2_w2, sa2_b2, fp1_w1, fp1_b1, fp1_w2, fp1_b2, fp2_w1, fp2_b1, fp2_w2, fp2_b2, fc_w, fc_b):
    raise NotImplementedError("write your pallas kernel here")


# speedup vs baseline: 12.2409x; 12.2409x over previous
import functools

import jax
import jax.numpy as jnp
from jax.experimental import pallas as pl
from jax.experimental.pallas import tpu as pltpu

F32 = jnp.float32
NEG_INF = float("-inf")
POS_INF = float("inf")


def _dotg(a, b, dims):
    return jax.lax.dot_general(a, b, (dims, ((), ())),
                               preferred_element_type=F32)


def _sa1_body(xT_ref, samp_ref, w1_ref, b1_ref, w2_ref, b2_ref,
              x1_ref, s1_ref, f2_s, d2_s, *, n_pts, q_chunk):
    xT = xT_ref[0]
    f2_s[...] = (_dotg(jax.nn.relu(_dotg(xT, w1_ref[...], ((0,), (0,)))
                                   + b1_ref[...]),
                       w2_ref[...], ((1,), (0,)))
                 + b2_ref[...])
    nn = jnp.sum(xT * xT, axis=0, keepdims=True)

    def chunk_body(c, carry):
        samp = samp_ref[0, pl.ds(c * q_chunk, q_chunk), :]
        iota = jax.lax.broadcasted_iota(jnp.int32, (q_chunk, n_pts),
                                        1).astype(F32)
        oh = (samp == iota).astype(F32)
        q = _dotg(oh, xT, ((1,), (1,)))
        s1_ref[0, pl.ds(c * q_chunk, q_chunk), :] = q
        qq = jnp.sum(q * q, axis=1, keepdims=True)
        qx = _dotg(q, xT, ((1,), (0,)))
        d2_s[...] = qq - 2.0 * qx + nn

        def pass_body(j, pc):
            m_prev, feats = pc
            d2v = d2_s[...]
            dm = jnp.where(d2v > m_prev, d2v, POS_INF)
            m = jnp.min(dm, axis=1, keepdims=True)
            sel = (d2v == m).astype(F32)
            row = _dotg(sel, f2_s[...], ((1,), (0,)))
            return m, jnp.maximum(feats, row)

        _, feats = jax.lax.fori_loop(
            0, 32, pass_body,
            (jnp.full((q_chunk, 1), NEG_INF, F32),
             jnp.full((q_chunk, 64), NEG_INF, F32)))
        x1_ref[0, pl.ds(c * q_chunk, q_chunk), :] = feats
        return carry

    jax.lax.fori_loop(0, 512 // q_chunk, chunk_body, 0)


def _sa2_fp1_body(x1_ref, samp_ref, s2w1_ref, s2b1_ref, s2w2_ref, s2b2_ref,
                  f1w1a_ref, f1w1b_ref, f1b1_ref, f1w2_ref, f1b2_ref,
                  x2u_ref):
    x1b = x1_ref[0]
    f2b = (_dotg(jax.nn.relu(_dotg(x1b, s2w1_ref[...], ((1,), (0,)))
                             + s2b1_ref[...]),
                 s2w2_ref[...], ((1,), (0,))) + s2b2_ref[...])
    samp = samp_ref[0]
    iota = jax.lax.broadcasted_iota(jnp.int32, (128, 512), 1).astype(F32)
    oh = (samp == iota).astype(F32)
    q2 = _dotg(oh, x1b, ((1,), (0,)))
    ones = jnp.ones((1, 64), F32)
    nn2 = _dotg(ones, x1b * x1b, ((1,), (1,)))
    qq2 = jnp.sum(q2 * q2, axis=1, keepdims=True)
    qx2 = _dotg(q2, x1b, ((1,), (1,)))
    d2b = qq2 - 2.0 * qx2 + nn2

    def pass_body(j, pc):
        m_prev, feats = pc
        dm = jnp.where(d2b > m_prev, d2b, POS_INF)
        m = jnp.min(dm, axis=1, keepdims=True)
        sel = (d2b == m).astype(F32)
        row = _dotg(sel, f2b, ((1,), (0,)))
        return m, jnp.maximum(feats, row)

    _, x2 = jax.lax.fori_loop(
        0, 32, pass_body,
        (jnp.full((128, 1), NEG_INF, F32),
         jnp.full((128, 128), NEG_INF, F32)))

    qqc = jnp.sum(x1b * x1b, axis=1, keepdims=True)
    nnc = _dotg(ones, q2 * q2, ((1,), (1,)))
    cross = _dotg(x1b, q2, ((1,), (1,)))
    d2c = qqc - 2.0 * cross + nnc
    m_prev = jnp.full((512, 1), NEG_INF, F32)
    for _ in range(3):
        dm = jnp.where(d2c > m_prev, d2c, POS_INF)
        m_prev = jnp.min(dm, axis=1, keepdims=True)
    a1 = jnp.where(d2c <= m_prev, jnp.float32(1.0 / 3.0), 0.0)
    interp = _dotg(a1, x2, ((1,), (0,)))
    h = jax.nn.relu(_dotg(x1b, f1w1a_ref[...], ((1,), (0,)))
                    + _dotg(interp, f1w1b_ref[...], ((1,), (0,)))
                    + f1b1_ref[...])
    x2u_ref[0] = _dotg(h, f1w2_ref[...], ((1,), (0,))) + f1b2_ref[...]


def _fp2_body(x8_ref, s1_ref, x2u_ref, w1a_ref, w1b_ref, b1_ref,
              w2_ref, b2_ref, fcw_ref, fcb_ref, out_ref):
    xc = x8_ref[0]
    s1b = s1_ref[0]
    ones = jnp.ones((1, 8), F32)
    nn = _dotg(ones, s1b * s1b, ((1,), (1,)))
    qq = jnp.sum(xc * xc, axis=1, keepdims=True)
    cross = _dotg(xc, s1b, ((1,), (1,)))
    d2 = qq - 2.0 * cross + nn
    m_prev = jnp.full((xc.shape[0], 1), NEG_INF, F32)
    for _ in range(3):
        dm = jnp.where(d2 > m_prev, d2, POS_INF)
        m_prev = jnp.min(dm, axis=1, keepdims=True)
    a2 = jnp.where(d2 <= m_prev, jnp.float32(1.0 / 3.0), 0.0)
    interp = _dotg(a2, x2u_ref[0], ((1,), (0,)))
    h = jax.nn.relu(_dotg(xc, w1a_ref[...], ((1,), (0,)))
                    + _dotg(interp, w1b_ref[...], ((1,), (0,)))
                    + b1_ref[...])
    xu = _dotg(h, w2_ref[...], ((1,), (0,))) + b2_ref[...]
    out_ref[0] = jax.nn.sigmoid(_dotg(xu, fcw_ref[...], ((1,), (0,)))
                                + fcb_ref[...])


def kernel(x, sa1_w1, sa1_b1, sa1_w2, sa1_b2, sa2_w1, sa2_b1, sa2_w2, sa2_b2,
           fp1_w1, fp1_b1, fp1_w2, fp1_b2, fp2_w1, fp2_b1, fp2_w2, fp2_b2,
           fc_w, fc_b):
    B, N, _ = x.shape
    q_chunk = 128
    row_chunk = 2048 if N % 2048 == 0 else N

    skey = jax.random.key(42)
    k1 = jax.random.split(jax.random.fold_in(skey, 1), B)
    k2 = jax.random.split(jax.random.fold_in(skey, 2), B)
    samp1 = jax.vmap(lambda k: jax.random.permutation(k, N)[:512])(k1)
    samp2 = jax.vmap(lambda k: jax.random.permutation(k, 512)[:128])(k2)
    samp1f = samp1.astype(F32).reshape(B, 512, 1)
    samp2f = samp2.astype(F32).reshape(B, 128, 1)

    x8 = jnp.pad(x, ((0, 0), (0, 0), (0, 5)))
    xT8 = jnp.transpose(x8, (0, 2, 1))
    sa1_w1p = jnp.pad(sa1_w1, ((0, 5), (0, 0)))
    fp2_w1a = jnp.pad(fp2_w1[:3], ((0, 5), (0, 0)))
    fp2_w1b = fp2_w1[3:]
    fp1_w1a = fp1_w1[:64]
    fp1_w1b = fp1_w1[64:]
    r1 = lambda v: v.reshape(1, -1)

    full = lambda shape: pl.BlockSpec(shape, lambda b, *_: (0,) * len(shape))

    x1, s1 = pl.pallas_call(
        functools.partial(_sa1_body, n_pts=N, q_chunk=q_chunk),
        grid=(B,),
        in_specs=[
            pl.BlockSpec((1, 8, N), lambda b: (b, 0, 0)),
            pl.BlockSpec((1, 512, 1), lambda b: (b, 0, 0)),
            full((8, 64)), full((1, 64)), full((64, 64)), full((1, 64)),
        ],
        out_specs=[
            pl.BlockSpec((1, 512, 64), lambda b: (b, 0, 0)),
            pl.BlockSpec((1, 512, 8), lambda b: (b, 0, 0)),
        ],
        out_shape=[
            jax.ShapeDtypeStruct((B, 512, 64), F32),
            jax.ShapeDtypeStruct((B, 512, 8), F32),
        ],
        scratch_shapes=[
            pltpu.VMEM((N, 64), F32),
            pltpu.VMEM((q_chunk, N), F32),
        ],
    )(xT8, samp1f, sa1_w1p, r1(sa1_b1), sa1_w2, r1(sa1_b2))

    x2u = pl.pallas_call(
        _sa2_fp1_body,
        grid=(B,),
        in_specs=[
            pl.BlockSpec((1, 512, 64), lambda b: (b, 0, 0)),
            pl.BlockSpec((1, 128, 1), lambda b: (b, 0, 0)),
            full((64, 128)), full((1, 128)), full((128, 128)), full((1, 128)),
            full((64, 64)), full((128, 64)), full((1, 64)),
            full((64, 64)), full((1, 64)),
        ],
        out_specs=pl.BlockSpec((1, 512, 64), lambda b: (b, 0, 0)),
        out_shape=jax.ShapeDtypeStruct((B, 512, 64), F32),
    )(x1, samp2f, sa2_w1, r1(sa2_b1), sa2_w2, r1(sa2_b2),
      fp1_w1a, fp1_w1b, r1(fp1_b1), fp1_w2, r1(fp1_b2))

    out = pl.pallas_call(
        _fp2_body,
        grid=(B, N // row_chunk),
        in_specs=[
            pl.BlockSpec((1, row_chunk, 8), lambda b, c: (b, c, 0)),
            pl.BlockSpec((1, 512, 8), lambda b, c: (b, 0, 0)),
            pl.BlockSpec((1, 512, 64), lambda b, c: (b, 0, 0)),
            full((8, 32)), full((64, 32)), full((1, 32)),
            full((32, 32)), full((1, 32)), full((32, 2)), full((1, 2)),
        ],
        out_specs=pl.BlockSpec((1, row_chunk, 2), lambda b, c: (b, c, 0)),
        out_shape=jax.ShapeDtypeStruct((B, N, 2), F32),
    )(x8, s1, x2u, fp2_w1a, fp2_w1b, r1(fp2_b1),
      fp2_w2, r1(fp2_b2), fc_w, r1(fc_b))

    return out
